# baseline (speedup 1.0000x reference)
import jax
import jax.numpy as jnp
from jax import lax
from jax.experimental import pallas as pl
from jax.experimental.pallas import tpu as pltpu
from jax.experimental.pallas import tpu_sc as plsc

N_NODES = 10000
N_EDGES = 320000
D_FEAT = 128
D_EDGE = 16
LANES = 16

C = 128
NCHUNK = N_EDGES // C
NCORES = 2
NSUB = 16
NW = NCORES * NSUB
CH_PER_W = (NCHUNK + NW - 1) // NW


def _sc_body(x_hbm, row_hbm, col_hbm, ea_hbm, w_hbm, b_hbm, out_hbm,
             row_v, col_v, ea_v, xr_v, xc_v, w_v, b_v, sem1, sem2):
    wid = lax.axis_index("c") * NSUB + lax.axis_index("s")
    pltpu.sync_copy(w_hbm, w_v)
    pltpu.sync_copy(b_hbm, b_v)

    def chunk_body(i, carry):
        cid = wid * CH_PER_W + i

        @pl.when(cid < NCHUNK)
        def _():
            base = cid * C
            pltpu.sync_copy(row_hbm.at[pl.ds(base, C)], row_v)
            pltpu.sync_copy(col_hbm.at[pl.ds(base, C)], col_v)
            pltpu.sync_copy(ea_hbm.at[pl.ds(base, C)], ea_v)
            cp1 = pltpu.async_copy(x_hbm.at[row_v], xr_v, sem1)
            cp2 = pltpu.async_copy(x_hbm.at[col_v], xc_v, sem2)
            cp1.wait()
            cp2.wait()
            for g in range(D_FEAT // LANES):
                gs = pl.ds(g * LANES, LANES)
                wv = [w_v[k, gs] for k in range(D_EDGE)]
                bg = b_v[gs]

                def e_body(e, ecarry):
                    acc = xr_v[e, gs] + xc_v[e, gs] + bg
                    eav = ea_v[e, :]
                    for k in range(D_EDGE):
                        acc = acc + eav[k] * wv[k]
                    xr_v[e, gs] = jnp.maximum(acc, 0.0)
                    return ecarry

                lax.fori_loop(0, C, e_body, 0)
            pltpu.sync_copy(xr_v, out_hbm.at[pl.ds(base, C)])

        return carry

    lax.fori_loop(0, CH_PER_W, chunk_body, 0)


def kernel(x, edge_index, edge_attr, W_e, b):
    row = edge_index[0]
    col = edge_index[1]
    mesh = plsc.VectorSubcoreMesh(core_axis_name="c", subcore_axis_name="s")
    f = pl.kernel(
        _sc_body,
        out_type=jax.ShapeDtypeStruct((N_EDGES, D_FEAT), jnp.float32),
        mesh=mesh,
        scratch_types=[
            pltpu.VMEM((C,), jnp.int32),
            pltpu.VMEM((C,), jnp.int32),
            pltpu.VMEM((C, D_EDGE), jnp.float32),
            pltpu.VMEM((C, D_FEAT), jnp.float32),
            pltpu.VMEM((C, D_FEAT), jnp.float32),
            pltpu.VMEM((D_EDGE, D_FEAT), jnp.float32),
            pltpu.VMEM((D_FEAT,), jnp.float32),
            pltpu.SemaphoreType.DMA,
            pltpu.SemaphoreType.DMA,
        ],
    )
    return f(x, row, col, edge_attr, W_e, b)


# speedup vs baseline: 3.4883x; 3.4883x over previous
import functools

import jax
import jax.numpy as jnp
from jax import lax
from jax.experimental import pallas as pl
from jax.experimental.pallas import tpu as pltpu
from jax.experimental.pallas import tpu_sc as plsc

N_NODES = 10000
N_EDGES = 320000
D_FEAT = 128
D_EDGE = 16
LANES = 16

C = 128
NCHUNK = N_EDGES // C
NCORES = 2
NSUB = 16
NW = NCORES * NSUB
CH_PER_W = (NCHUNK + NW - 1) // NW

BE = 6400


def _tc_matmul_body(ea_ref, w_ref, b_ref, out_ref):
    out_ref[...] = (
        jnp.dot(ea_ref[...], w_ref[...], preferred_element_type=jnp.float32)
        + b_ref[...]
    )


def _edge_feat_tc(edge_attr, W_e, b2d):
    return pl.pallas_call(
        _tc_matmul_body,
        grid=(N_EDGES // BE,),
        in_specs=[
            pl.BlockSpec((BE, D_EDGE), lambda i: (i, 0)),
            pl.BlockSpec((D_EDGE, D_FEAT), lambda i: (0, 0)),
            pl.BlockSpec((1, D_FEAT), lambda i: (0, 0)),
        ],
        out_specs=pl.BlockSpec((BE, D_FEAT), lambda i: (i, 0)),
        out_shape=jax.ShapeDtypeStruct((N_EDGES, D_FEAT), jnp.float32),
    )(edge_attr, W_e, b2d)


def _sc_body(x_hbm, row_hbm, col_hbm, ef_hbm, out_hbm,
             row_v, col_v, ef_v, xr_v, xc_v, sem1, sem2):
    wid = lax.axis_index("c") * NSUB + lax.axis_index("s")

    def chunk_body(i, carry):
        cid = wid * CH_PER_W + i

        @pl.when(cid < NCHUNK)
        def _():
            base = cid * C
            pltpu.sync_copy(row_hbm.at[pl.ds(base, C)], row_v)
            pltpu.sync_copy(col_hbm.at[pl.ds(base, C)], col_v)
            cp1 = pltpu.async_copy(x_hbm.at[row_v], xr_v, sem1)
            cp2 = pltpu.async_copy(x_hbm.at[col_v], xc_v, sem2)
            pltpu.sync_copy(ef_hbm.at[pl.ds(base, C)], ef_v)
            cp1.wait()
            cp2.wait()

            def e_body(e, ecarry):
                for g in range(D_FEAT // LANES):
                    gs = pl.ds(g * LANES, LANES)
                    acc = xr_v[e, gs] + xc_v[e, gs] + ef_v[e, gs]
                    xr_v[e, gs] = jnp.maximum(acc, 0.0)
                return ecarry

            lax.fori_loop(0, C, e_body, 0)
            pltpu.sync_copy(xr_v, out_hbm.at[pl.ds(base, C)])

        return carry

    lax.fori_loop(0, CH_PER_W, chunk_body, 0)


def kernel(x, edge_index, edge_attr, W_e, b):
    row = edge_index[0]
    col = edge_index[1]
    ef = _edge_feat_tc(edge_attr, W_e, b.reshape(1, D_FEAT))
    mesh = plsc.VectorSubcoreMesh(core_axis_name="c", subcore_axis_name="s")
    f = pl.kernel(
        _sc_body,
        out_type=jax.ShapeDtypeStruct((N_EDGES, D_FEAT), jnp.float32),
        mesh=mesh,
        scratch_types=[
            pltpu.VMEM((C,), jnp.int32),
            pltpu.VMEM((C,), jnp.int32),
            pltpu.VMEM((C, D_FEAT), jnp.float32),
            pltpu.VMEM((C, D_FEAT), jnp.float32),
            pltpu.VMEM((C, D_FEAT), jnp.float32),
            pltpu.SemaphoreType.DMA,
            pltpu.SemaphoreType.DMA,
        ],
    )
    return f(x, row, col, ef)


# speedup vs baseline: 4.8756x; 1.3977x over previous
import jax
import jax.numpy as jnp
from jax import lax
from jax.experimental import pallas as pl
from jax.experimental.pallas import tpu as pltpu
from jax.experimental.pallas import tpu_sc as plsc

N_NODES = 10000
N_EDGES = 320000
D_FEAT = 128
D_EDGE = 16
LANES = 16
NG = D_FEAT // LANES

C = 80
NCHUNK = N_EDGES // C
NCORES = 2
NSUB = 16
NW = NCORES * NSUB
CH_PER_W = NCHUNK // NW

BE = 6400


def _tc_matmul_body(ea_ref, w_ref, b_ref, out_ref):
    out_ref[...] = (
        jnp.dot(ea_ref[...], w_ref[...], preferred_element_type=jnp.float32)
        + b_ref[...]
    )


def _edge_feat_tc(edge_attr, W_e, b2d):
    return pl.pallas_call(
        _tc_matmul_body,
        grid=(N_EDGES // BE,),
        in_specs=[
            pl.BlockSpec((BE, D_EDGE), lambda i: (i, 0)),
            pl.BlockSpec((D_EDGE, D_FEAT), lambda i: (0, 0)),
            pl.BlockSpec((1, D_FEAT), lambda i: (0, 0)),
        ],
        out_specs=pl.BlockSpec((BE, D_FEAT), lambda i: (i, 0)),
        out_shape=jax.ShapeDtypeStruct((N_EDGES, D_FEAT), jnp.float32),
    )(edge_attr, W_e, b2d)


def _sc_body(x_hbm, row_hbm, col_hbm, ef_hbm, out_hbm,
             row0, row1, col0, col1, ef0, ef1, xr0, xr1, xc0, xc1,
             ov0, ov1, gs0, gs1, ws0, ws1):
    row_v = (row0, row1)
    col_v = (col0, col1)
    ef_v = (ef0, ef1)
    xr_v = (xr0, xr1)
    xc_v = (xc0, xc1)
    out_v = (ov0, ov1)
    gsem = (gs0, gs1)
    wsem = (ws0, ws1)

    wid = lax.axis_index("c") * NSUB + lax.axis_index("s")
    wbase = wid * CH_PER_W

    def start(cid, b):
        base = cid * C
        pltpu.sync_copy(row_hbm.at[pl.ds(base, C)], row_v[b])
        pltpu.sync_copy(col_hbm.at[pl.ds(base, C)], col_v[b])
        pltpu.async_copy(x_hbm.at[row_v[b]], xr_v[b], gsem[b])
        pltpu.async_copy(x_hbm.at[col_v[b]], xc_v[b], gsem[b])
        pltpu.async_copy(ef_hbm.at[pl.ds(base, C)], ef_v[b], gsem[b])

    def wait_gathers(cid, b):
        base = cid * C
        pltpu.make_async_copy(x_hbm.at[row_v[b]], xr_v[b], gsem[b]).wait()
        pltpu.make_async_copy(x_hbm.at[col_v[b]], xc_v[b], gsem[b]).wait()
        pltpu.make_async_copy(ef_hbm.at[pl.ds(base, C)], ef_v[b], gsem[b]).wait()

    def compute(b):
        def e_body(e, ecarry):
            for g in range(NG):
                gsl = pl.ds(g * LANES, LANES)
                acc = xr_v[b][e, gsl] + xc_v[b][e, gsl] + ef_v[b][e, gsl]
                out_v[b][e, gsl] = jnp.maximum(acc, 0.0)
            return ecarry

        lax.fori_loop(0, C, e_body, 0)

    def write(cid, b):
        pltpu.async_copy(out_v[b], out_hbm.at[pl.ds(cid * C, C)], wsem[b])

    def wait_write(cid, b):
        pltpu.make_async_copy(out_v[b], out_hbm.at[pl.ds(cid * C, C)],
                              wsem[b]).wait()

    start(wbase + 0, 0)
    start(wbase + 1, 1)

    def pair_body(j, carry):
        i0 = 2 * j
        for b in range(2):
            cid = wbase + i0 + b
            wait_gathers(cid, b)

            @pl.when(j >= 1)
            def _():
                wait_write(cid - 2, b)

            compute(b)
            write(cid, b)

            @pl.when(i0 + b + 2 < CH_PER_W)
            def _():
                start(cid + 2, b)

        return carry

    lax.fori_loop(0, (CH_PER_W - 1) // 2, pair_body, 0)

    cid = wbase + CH_PER_W - 1
    wait_gathers(cid, 0)
    wait_write(cid - 2, 0)
    compute(0)
    write(cid, 0)
    wait_write(cid, 0)
    wait_write(cid - 1, 1)


def kernel(x, edge_index, edge_attr, W_e, b):
    row = edge_index[0]
    col = edge_index[1]
    ef = _edge_feat_tc(edge_attr, W_e, b.reshape(1, D_FEAT))
    mesh = plsc.VectorSubcoreMesh(core_axis_name="c", subcore_axis_name="s")
    f = pl.kernel(
        _sc_body,
        out_type=jax.ShapeDtypeStruct((N_EDGES, D_FEAT), jnp.float32),
        mesh=mesh,
        scratch_types=[
            pltpu.VMEM((C,), jnp.int32),
            pltpu.VMEM((C,), jnp.int32),
            pltpu.VMEM((C,), jnp.int32),
            pltpu.VMEM((C,), jnp.int32),
            pltpu.VMEM((C, D_FEAT), jnp.float32),
            pltpu.VMEM((C, D_FEAT), jnp.float32),
            pltpu.VMEM((C, D_FEAT), jnp.float32),
            pltpu.VMEM((C, D_FEAT), jnp.float32),
            pltpu.VMEM((C, D_FEAT), jnp.float32),
            pltpu.VMEM((C, D_FEAT), jnp.float32),
            pltpu.VMEM((C, D_FEAT), jnp.float32),
            pltpu.VMEM((C, D_FEAT), jnp.float32),
            pltpu.SemaphoreType.DMA,
            pltpu.SemaphoreType.DMA,
            pltpu.SemaphoreType.DMA,
            pltpu.SemaphoreType.DMA,
        ],
    )
    return f(x, row, col, ef)
